# baseline (speedup 1.0000x reference)
import functools

import jax
import jax.numpy as jnp
from jax import lax
from jax.experimental import pallas as pl
from jax.experimental.pallas import tpu as pltpu
from jax.experimental.pallas import tpu_sc as plsc

PE_SIZE = 2048
N_POS = 16384

_NC = 2
_NS = 16
_NW = _NC * _NS
_ROWS_W = N_POS // _NW
_WORDS_W = _ROWS_W * PE_SIZE
_ZBUF = 65536
_NZDMA = _WORDS_W // _ZBUF
_SCHUNK = 128
_NSCH = _ROWS_W // _SCHUNK


def _body(pos_hbm, zeros_hbm, out_hbm, idx_v, fidx_v, ones_v, zbuf_v,
          zsem, ssem):
    wid = lax.axis_index("s") * _NC + lax.axis_index("c")
    base_row = wid * _ROWS_W
    base_word = wid * _WORDS_W

    pltpu.sync_copy(pos_hbm.at[pl.ds(base_row, _ROWS_W)], idx_v)
    pltpu.sync_copy(zeros_hbm, zbuf_v)

    copies = []
    for i in range(_NZDMA):
        dst = out_hbm.at[pl.ds(pl.multiple_of(base_word + i * _ZBUF, _ZBUF),
                               _ZBUF)]
        copies.append(pltpu.async_copy(zbuf_v, dst, zsem))

    iota16 = lax.broadcasted_iota(jnp.int32, (16,), 0)
    for j in range(_NSCH):
        for t in range(_SCHUNK // 16):
            r0 = j * _SCHUNK + t * 16
            idx16 = idx_v[pl.ds(r0, 16)]
            clamped = jnp.minimum(idx16, PE_SIZE - 1)
            fidx_v[j, pl.ds(t * 16, 16)] = (
                (base_row + r0 + iota16) * PE_SIZE + clamped)
    for t in range(_SCHUNK // 16):
        ones_v[pl.ds(t * 16, 16)] = jnp.full((16,), 1, jnp.int32)

    for c in copies:
        c.wait()

    for j in range(_NSCH):
        pltpu.async_copy(ones_v, out_hbm.at[fidx_v.at[j]], ssem).wait()


@functools.partial(
    pl.kernel,
    out_type=jax.ShapeDtypeStruct((N_POS * PE_SIZE,), jnp.int32),
    mesh=plsc.VectorSubcoreMesh(core_axis_name="c", subcore_axis_name="s"),
    scratch_types=[
        pltpu.VMEM((_ROWS_W,), jnp.int32),
        pltpu.VMEM((_NSCH, _SCHUNK), jnp.int32),
        pltpu.VMEM((_SCHUNK,), jnp.int32),
        pltpu.VMEM((_ZBUF,), jnp.int32),
        pltpu.SemaphoreType.DMA,
        pltpu.SemaphoreType.DMA,
    ],
)
def _onehot_sc(pos_hbm, zeros_hbm, out_hbm, idx_v, fidx_v, ones_v, zbuf_v,
               zsem, ssem):
    _body(pos_hbm, zeros_hbm, out_hbm, idx_v, fidx_v, ones_v, zbuf_v,
          zsem, ssem)


def kernel(position):
    pos_flat = position.reshape(N_POS)
    zeros = jnp.zeros((_ZBUF,), jnp.int32)
    out_flat = _onehot_sc(pos_flat, zeros)
    return out_flat.reshape(N_POS, PE_SIZE)


# speedup vs baseline: 2.8038x; 2.8038x over previous
import functools

import jax
import jax.numpy as jnp
from jax import lax
from jax.experimental import pallas as pl
from jax.experimental.pallas import tpu as pltpu
from jax.experimental.pallas import tpu_sc as plsc

PE_SIZE = 2048
N_POS = 16384

_NC = 2
_NS = 16
_NW = _NC * _NS
_ROWS_W = N_POS // _NW
_BR = 16
_NBLK = _ROWS_W // _BR


def _body(pos_hbm, zeros_hbm, out_hbm, idx_v, zb0, zb1, sem0, sem1):
    wid = lax.axis_index("s") * _NC + lax.axis_index("c")
    base_row = wid * _ROWS_W

    pltpu.sync_copy(pos_hbm.at[pl.ds(base_row, _ROWS_W)], idx_v)
    pltpu.sync_copy(zeros_hbm, zb0)
    pltpu.sync_copy(zeros_hbm, zb1)

    bufs = (zb0, zb1)
    sems = (sem0, sem1)
    iota16 = lax.broadcasted_iota(jnp.int32, (16,), 0)
    ones16 = jnp.full((16,), 1, jnp.int32)
    zeros16 = jnp.zeros((16,), jnp.int32)

    handles = [None, None]
    dirty = [None, None]
    for it in range(_NBLK):
        b = it % 2
        if handles[b] is not None:
            handles[b].wait()
            plsc.store_scatter(bufs[b], [iota16, dirty[b]], zeros16)
        idx16 = idx_v[pl.ds(it * _BR, 16)]
        col16 = jnp.minimum(idx16, PE_SIZE - 1)
        plsc.store_scatter(bufs[b], [iota16, col16], ones16)
        handles[b] = pltpu.async_copy(
            bufs[b], out_hbm.at[pl.ds(base_row + it * _BR, _BR), :], sems[b])
        dirty[b] = col16
    handles[0].wait()
    handles[1].wait()


@functools.partial(
    pl.kernel,
    out_type=jax.ShapeDtypeStruct((N_POS, PE_SIZE), jnp.int32),
    mesh=plsc.VectorSubcoreMesh(core_axis_name="c", subcore_axis_name="s"),
    compiler_params=pltpu.CompilerParams(needs_layout_passes=False),
    scratch_types=[
        pltpu.VMEM((_ROWS_W,), jnp.int32),
        pltpu.VMEM((_BR, PE_SIZE), jnp.int32),
        pltpu.VMEM((_BR, PE_SIZE), jnp.int32),
        pltpu.SemaphoreType.DMA,
        pltpu.SemaphoreType.DMA,
    ],
)
def _onehot_sc(pos_hbm, zeros_hbm, out_hbm, idx_v, zb0, zb1, sem0, sem1):
    _body(pos_hbm, zeros_hbm, out_hbm, idx_v, zb0, zb1, sem0, sem1)


def kernel(position):
    pos_flat = position.reshape(N_POS)
    zeros = jnp.zeros((_BR, PE_SIZE), jnp.int32)
    return _onehot_sc(pos_flat, zeros)


# speedup vs baseline: 3.2570x; 1.1616x over previous
import functools

import jax
import jax.numpy as jnp
from jax import lax
from jax.experimental import pallas as pl
from jax.experimental.pallas import tpu as pltpu
from jax.experimental.pallas import tpu_sc as plsc

PE_SIZE = 2048
N_POS = 16384

_NC = 2
_NS = 16
_NW = _NC * _NS
_ROWS_W = N_POS // _NW
_BR = 16
_NBLK = _ROWS_W // _BR


def _body(pos_hbm, zeros_hbm, out_hbm, idx_v, zb0, zb1, sem0, sem1):
    wid = lax.axis_index("s") * _NC + lax.axis_index("c")
    base_row = wid * _ROWS_W

    pltpu.sync_copy(pos_hbm.at[pl.ds(base_row, _ROWS_W)], idx_v)
    pltpu.sync_copy(zeros_hbm.at[wid], zb0)

    bufs = (zb0, zb1)
    sems = (sem0, sem1)
    iota16 = lax.broadcasted_iota(jnp.int32, (16,), 0)
    ones16 = jnp.full((16,), 1, jnp.int32)
    zeros16 = jnp.zeros((16,), jnp.int32)

    handles = [None, None]
    dirty = [None, None]
    for it in range(_NBLK):
        b = it % 2
        if it == 1:
            pltpu.sync_copy(zeros_hbm.at[wid], zb1)
        if handles[b] is not None:
            handles[b].wait()
            plsc.store_scatter(bufs[b], [iota16, dirty[b]], zeros16)
        idx16 = idx_v[pl.ds(it * _BR, 16)]
        col16 = jnp.minimum(idx16, PE_SIZE - 1)
        plsc.store_scatter(bufs[b], [iota16, col16], ones16)
        handles[b] = pltpu.async_copy(
            bufs[b], out_hbm.at[pl.ds(base_row + it * _BR, _BR), :], sems[b])
        dirty[b] = col16
    handles[0].wait()
    handles[1].wait()


@functools.partial(
    pl.kernel,
    out_type=jax.ShapeDtypeStruct((N_POS, PE_SIZE), jnp.int32),
    mesh=plsc.VectorSubcoreMesh(core_axis_name="c", subcore_axis_name="s"),
    compiler_params=pltpu.CompilerParams(needs_layout_passes=False),
    scratch_types=[
        pltpu.VMEM((_ROWS_W,), jnp.int32),
        pltpu.VMEM((_BR, PE_SIZE), jnp.int32),
        pltpu.VMEM((_BR, PE_SIZE), jnp.int32),
        pltpu.SemaphoreType.DMA,
        pltpu.SemaphoreType.DMA,
    ],
)
def _onehot_sc(pos_hbm, zeros_hbm, out_hbm, idx_v, zb0, zb1, sem0, sem1):
    _body(pos_hbm, zeros_hbm, out_hbm, idx_v, zb0, zb1, sem0, sem1)


def kernel(position):
    pos_flat = position.reshape(N_POS)
    zeros = jnp.zeros((_NW, _BR, PE_SIZE), jnp.int32)
    return _onehot_sc(pos_flat, zeros)


# speedup vs baseline: 3.3633x; 1.0326x over previous
import functools

import jax
import jax.numpy as jnp
from jax import lax
from jax.experimental import pallas as pl
from jax.experimental.pallas import tpu as pltpu
from jax.experimental.pallas import tpu_sc as plsc

PE_SIZE = 2048
N_POS = 16384

_NC = 2
_NS = 16
_NW = _NC * _NS
_ROWS_W = N_POS // _NW
_BR = 16
_NBLK = _ROWS_W // _BR


def _body(pos_hbm, zeros_hbm, out_hbm, idx_v, zb0, zb1, zsh, sem0, sem1):
    s = lax.axis_index("s")
    wid = s * _NC + lax.axis_index("c")
    base_row = wid * _ROWS_W

    pltpu.sync_copy(pos_hbm.at[pl.ds(base_row, _ROWS_W)], idx_v)

    @pl.when(s == 0)
    def _():
        pltpu.sync_copy(zeros_hbm, zsh)

    plsc.subcore_barrier()
    pltpu.sync_copy(zsh, zb0)

    bufs = (zb0, zb1)
    sems = (sem0, sem1)
    iota16 = lax.broadcasted_iota(jnp.int32, (16,), 0)
    ones16 = jnp.full((16,), 1, jnp.int32)
    zeros16 = jnp.zeros((16,), jnp.int32)

    handles = [None, None]
    dirty = [None, None]
    for it in range(_NBLK):
        b = it % 2
        if it == 1:
            pltpu.sync_copy(zsh, zb1)
        if handles[b] is not None:
            handles[b].wait()
            plsc.store_scatter(bufs[b], [iota16, dirty[b]], zeros16)
        idx16 = idx_v[pl.ds(it * _BR, 16)]
        col16 = jnp.minimum(idx16, PE_SIZE - 1)
        plsc.store_scatter(bufs[b], [iota16, col16], ones16)
        handles[b] = pltpu.async_copy(
            bufs[b], out_hbm.at[pl.ds(base_row + it * _BR, _BR), :], sems[b])
        dirty[b] = col16
    handles[0].wait()
    handles[1].wait()


@functools.partial(
    pl.kernel,
    out_type=jax.ShapeDtypeStruct((N_POS, PE_SIZE), jnp.int32),
    mesh=plsc.VectorSubcoreMesh(core_axis_name="c", subcore_axis_name="s"),
    compiler_params=pltpu.CompilerParams(needs_layout_passes=False),
    scratch_types=[
        pltpu.VMEM((_ROWS_W,), jnp.int32),
        pltpu.VMEM((_BR, PE_SIZE), jnp.int32),
        pltpu.VMEM((_BR, PE_SIZE), jnp.int32),
        pltpu.VMEM_SHARED((_BR, PE_SIZE), jnp.int32),
        pltpu.SemaphoreType.DMA,
        pltpu.SemaphoreType.DMA,
    ],
)
def _onehot_sc(pos_hbm, zeros_hbm, out_hbm, idx_v, zb0, zb1, zsh, sem0, sem1):
    _body(pos_hbm, zeros_hbm, out_hbm, idx_v, zb0, zb1, zsh, sem0, sem1)


def kernel(position):
    pos_flat = position.reshape(N_POS)
    zeros = jnp.zeros((_BR, PE_SIZE), jnp.int32)
    return _onehot_sc(pos_flat, zeros)


# speedup vs baseline: 3.4440x; 1.0240x over previous
import functools

import jax
import jax.numpy as jnp
from jax import lax
from jax.experimental import pallas as pl
from jax.experimental.pallas import tpu as pltpu
from jax.experimental.pallas import tpu_sc as plsc

PE_SIZE = 2048
N_POS = 16384

_NC = 2
_NS = 16
_NW = _NC * _NS
_ROWS_W = N_POS // _NW
_BR = 16
_NBLK = _ROWS_W // _BR


def _body(pos_hbm, zeros_hbm, out_hbm, idx_v, zb0, zb1, zsh, sem0, sem1):
    s = lax.axis_index("s")
    wid = s * _NC + lax.axis_index("c")
    base_row = wid * _ROWS_W

    pltpu.sync_copy(pos_hbm.at[pl.ds(base_row, _ROWS_W)], idx_v)

    @pl.when(s == 0)
    def _():
        pltpu.sync_copy(zeros_hbm, zsh)

    plsc.subcore_barrier()
    pltpu.sync_copy(zsh, zb0)

    bufs = (zb0, zb1)
    sems = (sem0, sem1)
    iota16 = lax.broadcasted_iota(jnp.int32, (16,), 0)
    ones16 = jnp.full((16,), 1, jnp.int32)
    zeros16 = jnp.zeros((16,), jnp.int32)

    def _block(it, b, dprev):
        if dprev is not None:
            pltpu.make_async_copy(
                bufs[b], out_hbm.at[pl.ds(0, _BR), :], sems[b]).wait()
            plsc.store_scatter(bufs[b], [iota16, dprev], zeros16)
        idx16 = idx_v[pl.ds(it * _BR, 16)]
        col16 = jnp.minimum(idx16, PE_SIZE - 1)
        plsc.store_scatter(bufs[b], [iota16, col16], ones16)
        pltpu.async_copy(
            bufs[b], out_hbm.at[pl.ds(base_row + it * _BR, _BR), :], sems[b])
        return col16

    d0 = _block(0, 0, None)
    pltpu.sync_copy(zsh, zb1)
    d1 = _block(1, 1, None)

    def _pair(p, dirty):
        d0 = _block(p * 2, 0, dirty[0])
        d1 = _block(p * 2 + 1, 1, dirty[1])
        return (d0, d1)

    lax.fori_loop(1, _NBLK // 2, _pair, (d0, d1))

    pltpu.make_async_copy(zb0, out_hbm.at[pl.ds(0, _BR), :], sem0).wait()
    pltpu.make_async_copy(zb1, out_hbm.at[pl.ds(0, _BR), :], sem1).wait()


@functools.partial(
    pl.kernel,
    out_type=jax.ShapeDtypeStruct((N_POS, PE_SIZE), jnp.int32),
    mesh=plsc.VectorSubcoreMesh(core_axis_name="c", subcore_axis_name="s"),
    compiler_params=pltpu.CompilerParams(
        needs_layout_passes=False,
        disable_bounds_checks=True,
        disable_semaphore_checks=True,
    ),
    scratch_types=[
        pltpu.VMEM((_ROWS_W,), jnp.int32),
        pltpu.VMEM((_BR, PE_SIZE), jnp.int32),
        pltpu.VMEM((_BR, PE_SIZE), jnp.int32),
        pltpu.VMEM_SHARED((_BR, PE_SIZE), jnp.int32),
        pltpu.SemaphoreType.DMA,
        pltpu.SemaphoreType.DMA,
    ],
)
def _onehot_sc(pos_hbm, zeros_hbm, out_hbm, idx_v, zb0, zb1, zsh, sem0, sem1):
    _body(pos_hbm, zeros_hbm, out_hbm, idx_v, zb0, zb1, zsh, sem0, sem1)


def kernel(position):
    pos_flat = position.reshape(N_POS)
    zeros = jnp.zeros((_BR, PE_SIZE), jnp.int32)
    return _onehot_sc(pos_flat, zeros)
